# baseline (speedup 1.0000x reference)
import functools
import math

import jax
import jax.numpy as jnp
from jax import lax
from jax.experimental import pallas as pl
from jax.experimental.pallas import tpu as pltpu
from jax.experimental.pallas import tpu_sc as plsc

N = 50000
E = 800000
INPUT_DIM = 16
MUL = 16
NB = 10
MAX_RADIUS = 3.5
NUM_NEI = 16
HID = 72
DIMS = [MUL, HID, HID, HID, 16]
NL = 4
EDGE_ATTR_DIM = 13

SLICE = 32
HPAD = 96
NSLICES = [3, 3, 3, 1]
POFF = [0, 3, 6, 9]
NPLANES = 10

CHUNK = 128
NCHUNKS = E // CHUNK
NWORK = 32
CPW = (NCHUNKS + NWORK - 1) // NWORK
RPT = N // 16
FCH = 125
NFL = RPT // FCH

EBLK = 1600
NBLK = 2000


def _sc_edge_vec(pos16, src, dst):
    mesh = plsc.VectorSubcoreMesh(core_axis_name="c", subcore_axis_name="s")

    @functools.partial(
        pl.kernel,
        out_type=jax.ShapeDtypeStruct((E, 16), jnp.float32),
        mesh=mesh,
        compiler_params=pltpu.CompilerParams(use_tc_tiling_on_sc=False),
        scratch_types=[
            pltpu.VMEM((CHUNK,), jnp.int32),
            pltpu.VMEM((CHUNK,), jnp.int32),
            pltpu.VMEM((CHUNK, 16), jnp.float32),
            pltpu.VMEM((CHUNK, 16), jnp.float32),
            pltpu.SemaphoreType.DMA,
            pltpu.SemaphoreType.DMA,
        ],
    )
    def k(pos_ref, src_ref, dst_ref, vec_ref, sidx, didx, pb, qb, sem1, sem2):
        wid = lax.axis_index("s") * 2 + lax.axis_index("c")

        def chunk(i, carry):
            c = wid + NWORK * i

            @pl.when(c < NCHUNKS)
            def _():
                base = c * CHUNK
                pltpu.sync_copy(src_ref.at[pl.ds(base, CHUNK)], sidx)
                pltpu.sync_copy(dst_ref.at[pl.ds(base, CHUNK)], didx)
                cp1 = pltpu.async_copy(pos_ref.at[sidx], pb, sem1)
                cp2 = pltpu.async_copy(pos_ref.at[didx], qb, sem2)
                cp1.wait()
                cp2.wait()

                def sub(j, cc):
                    pb[j, pl.ds(0, 16)] = pb[j, pl.ds(0, 16)] - qb[j, pl.ds(0, 16)]
                    return cc

                lax.fori_loop(0, CHUNK, sub, 0)
                pltpu.sync_copy(pb, vec_ref.at[pl.ds(base, CHUNK)])

            return carry

        lax.fori_loop(0, CPW, chunk, 0)

    return k(pos16, src, dst)


G = 128
SUB = G // CHUNK
NGR = E // G
IPW = (NGR + NWORK - 1) // NWORK
OUTER = IPW // 2


def _sc_message(l, src, dst, p_flat, hws):
    S = NSLICES[l]
    mesh = plsc.VectorSubcoreMesh(core_axis_name="c", subcore_axis_name="s")
    scratch = [
        [pltpu.VMEM((G,), jnp.int32) for _ in range(2)],
        [pltpu.VMEM((SUB, CHUNK), jnp.int32) for _ in range(2)],
        [pltpu.VMEM((SUB, CHUNK), jnp.int32) for _ in range(2)],
        [pltpu.VMEM((G, SLICE), jnp.float32) for _ in range(2)],
        [pltpu.VMEM((G, SLICE), jnp.float32) for _ in range(2)],
        [pltpu.VMEM((G, SLICE), jnp.float32) for _ in range(2)],
        pltpu.VMEM((FCH, SLICE), jnp.float32),
        pltpu.VMEM_SHARED((N, SLICE), jnp.float32),
        [pltpu.SemaphoreType.DMA for _ in range(2)],
        [pltpu.SemaphoreType.DMA for _ in range(2)],
        pltpu.SemaphoreType.DMA,
    ]

    @functools.partial(
        pl.kernel,
        out_type=tuple(jax.ShapeDtypeStruct((2 * N, SLICE), jnp.float32)
                       for _ in range(S)),
        mesh=mesh,
        compiler_params=pltpu.CompilerParams(use_tc_tiling_on_sc=False),
        scratch_types=scratch,
    )
    def k(src_ref, dst_ref, p_ref, *rest):
        hw_refs = rest[:S]
        agg_refs = rest[S:2 * S]
        (sidx, didx, sdix, hb, pb, mb, zb, acc, gsem, ssem, fsem) = rest[2 * S:]
        cid = lax.axis_index("c")
        tid = lax.axis_index("s")
        wid = tid * 2 + cid

        def z(j, c):
            zb[j, pl.ds(0, 16)] = jnp.zeros((16,), jnp.float32)
            zb[j, pl.ds(16, 16)] = jnp.zeros((16,), jnp.float32)
            return c

        lax.fori_loop(0, FCH, z, 0)

        for s in range(S):
            hw_ref = hw_refs[s]
            prow0 = (POFF[l] + s) * E

            for f0 in range(0, NFL, 5):
                zcps = [pltpu.async_copy(
                    zb, acc.at[pl.ds(tid * RPT + f * FCH, FCH)], fsem)
                    for f in range(f0, min(NFL, f0 + 5))]
                for cp in zcps:
                    cp.wait()
            plsc.subcore_barrier()

            def issue(i, b):
                c = wid + NWORK * i

                @pl.when(c < NGR)
                def _():
                    base = c * G
                    pltpu.sync_copy(src_ref.at[pl.ds(base, G)], sidx[b])
                    for j in range(SUB):
                        pltpu.sync_copy(
                            dst_ref.at[pl.ds(base + CHUNK * j, CHUNK)],
                            didx[b].at[j])
                    for j in range(SUB):
                        pltpu.async_copy(
                            hw_ref.at[sidx[b].at[pl.ds(CHUNK * j, CHUNK)]],
                            hb[b].at[pl.ds(CHUNK * j, CHUNK)], gsem[b])
                    pltpu.async_copy(p_ref.at[pl.ds(prow0 + base, G)],
                                     pb[b], gsem[b])

            def process(i, i2, b):
                c = wid + NWORK * i

                @pl.when(c < NGR)
                def _():
                    for j in range(SUB):
                        pltpu.make_async_copy(
                            hw_ref.at[sidx[b].at[pl.ds(CHUNK * j, CHUNK)]],
                            hb[b].at[pl.ds(CHUNK * j, CHUNK)], gsem[b]).wait()
                    pltpu.make_async_copy(p_ref.at[pl.ds(0, G)],
                                          pb[b], gsem[b]).wait()

                    @pl.when(i2 >= 1)
                    def _():
                        for j in range(SUB):
                            pltpu.make_async_copy(
                                mb[b].at[pl.ds(CHUNK * j, CHUNK)],
                                acc.at[sdix[b].at[j]], ssem[b]).wait()

                    for j in range(SUB):
                        for t in range(CHUNK // 16):
                            sdix[b][j, pl.ds(16 * t, 16)] = \
                                didx[b][j, pl.ds(16 * t, 16)]

                    def mul(jj, cc):
                        for r in range(4):
                            row = jj * 4 + r
                            mb[b][row, pl.ds(0, 16)] = \
                                pb[b][row, pl.ds(0, 16)] * hb[b][row, pl.ds(0, 16)]
                            mb[b][row, pl.ds(16, 16)] = \
                                pb[b][row, pl.ds(16, 16)] * hb[b][row, pl.ds(16, 16)]
                        return cc

                    lax.fori_loop(0, G // 4, mul, 0)

                    for j in range(SUB):
                        pltpu.async_copy(
                            mb[b].at[pl.ds(CHUNK * j, CHUNK)],
                            acc.at[sdix[b].at[j]], ssem[b], add=True)

            issue(0, 0)
            issue(1, 1)

            def outer(i2, carry):
                for b in range(2):
                    i = i2 * 2 + b
                    process(i, i2, b)
                    issue(i + 2, b)
                return carry

            lax.fori_loop(0, OUTER, outer, 0)

            for b in range(2):
                for j in range(SUB):
                    pltpu.make_async_copy(
                        mb[b].at[pl.ds(CHUNK * j, CHUNK)],
                        acc.at[sdix[b].at[j]], ssem[b]).wait()

            plsc.subcore_barrier()

            for f0 in range(0, NFL, 5):
                fcps = [pltpu.async_copy(
                    acc.at[pl.ds(tid * RPT + f * FCH, FCH)],
                    agg_refs[s].at[pl.ds(cid * N + tid * RPT + f * FCH, FCH)],
                    fsem) for f in range(f0, min(NFL, f0 + 5))]
                for cp in fcps:
                    cp.wait()
            plsc.subcore_barrier()

    return k(src, dst, p_flat, *hws)


def _edge_dense_kernel(vec_ref, ea_ref, f1_ref, f2_ref, f3_ref, we_ref,
                       eaf_ref, emb_ref, p_ref):
    vec = vec_ref[...][:, 0:3]
    r2 = jnp.sum(vec * vec, axis=1, keepdims=True) + 1e-12
    r = jnp.sqrt(r2)
    u = vec / r
    x, y, z = u[:, 0:1], u[:, 1:2], u[:, 2:3]
    s3 = math.sqrt(3.0)
    s15 = math.sqrt(15.0)
    s5h = math.sqrt(5.0) / 2.0
    zero = jnp.zeros_like(x)
    eaf16 = jnp.concatenate([
        ea_ref[...],
        jnp.ones_like(x), s3 * x, s3 * y, s3 * z,
        s15 * x * y, s15 * y * z, s5h * (3.0 * z * z - 1.0),
        s15 * x * z, (s15 / 2.0) * (x * x - y * y),
        zero, zero, zero,
    ], axis=1)
    eaf_ref[...] = eaf16
    eaf = eaf16[:, 0:EDGE_ATTR_DIM]

    step = MAX_RADIUS / (NB + 1)
    values = (lax.broadcasted_iota(jnp.int32, (1, NB), 1)
              .astype(jnp.float32) + 1.0) * step
    diff = (r - values) / step
    theta = jnp.reshape(r, (1, r.shape[0])) * (jnp.pi / 2.0 / step)
    ct = jnp.reshape(jnp.cos(theta), (r.shape[0], 1))
    st = jnp.reshape(jnp.sin(theta), (r.shape[0], 1))
    m4 = jnp.remainder(lax.broadcasted_iota(jnp.int32, (1, NB), 1) + 1, 4)
    ck = jnp.where(m4 == 0, 1.0, 0.0) + jnp.where(m4 == 2, -1.0, 0.0)
    sk = jnp.where(m4 == 1, 1.0, 0.0) + jnp.where(m4 == 3, -1.0, 0.0)
    emb = (ct * ck + st * sk) * (diff > -1.0) * (diff < 1.0)
    emb = emb * (NB ** 0.5)
    emb_ref[...] = jnp.concatenate([emb, jnp.zeros_like(emb[:, 0:6])], axis=1)

    e1 = jax.nn.gelu(jnp.dot(emb, f1_ref[...],
                             preferred_element_type=jnp.float32)
                     / math.sqrt(float(NB)))
    e2 = jax.nn.gelu(jnp.dot(e1, f2_ref[...],
                             preferred_element_type=jnp.float32)
                     / math.sqrt(float(MUL)))
    w_all = jnp.dot(e2, f3_ref[...], preferred_element_type=jnp.float32) \
        / math.sqrt(float(2 * MUL))
    a_all = jnp.dot(eaf, we_ref[...], preferred_element_type=jnp.float32) \
        / math.sqrt(float(EDGE_ATTR_DIM))
    p_all = w_all * a_all * (1.0 / math.sqrt(float(NUM_NEI)))
    for l in range(NL):
        for s in range(NSLICES[l]):
            c0 = HPAD * l + SLICE * s
            p_ref[POFF[l] + s] = p_all[:, c0:c0 + SLICE]


def _edge_dense(vec16, ea, f1s, f2s, f3s, wes):
    grid = E // EBLK
    eb = lambda d: pl.BlockSpec((EBLK, d), lambda i: (i, 0))
    full = lambda a: pl.BlockSpec(a.shape, lambda i: (0,) * a.ndim)
    return pl.pallas_call(
        _edge_dense_kernel,
        grid=(grid,),
        in_specs=[eb(16), eb(4), full(f1s), full(f2s), full(f3s), full(wes)],
        out_specs=(eb(16), eb(16),
                   pl.BlockSpec((NPLANES, EBLK, SLICE), lambda i: (0, i, 0))),
        out_shape=(
            jax.ShapeDtypeStruct((E, 16), jnp.float32),
            jax.ShapeDtypeStruct((E, 16), jnp.float32),
            jax.ShapeDtypeStruct((NPLANES, E, SLICE), jnp.float32),
        ),
    )(vec16, ea, f1s, f2s, f3s, wes)


def _c0_kernel(x_ref, w1_ref, w2_ref, wm_ref, h0_ref, hw0_ref, hw1_ref, hw2_ref):
    h0 = jax.nn.gelu(jnp.dot(x_ref[...], w1_ref[...],
                             preferred_element_type=jnp.float32))
    h0 = jnp.dot(h0, w2_ref[...], preferred_element_type=jnp.float32)
    h0_ref[...] = h0
    hw = jnp.dot(h0, wm_ref[...], preferred_element_type=jnp.float32) * 0.25
    hw0_ref[...] = hw[:, 0:SLICE]
    hw1_ref[...] = hw[:, SLICE:2 * SLICE]
    hw2_ref[...] = hw[:, 2 * SLICE:3 * SLICE]


def _c0(x, w1f, w2f, wm0f):
    grid = N // NBLK
    nb = lambda d: pl.BlockSpec((NBLK, d), lambda i: (i, 0))
    full = lambda a: pl.BlockSpec(a.shape, lambda i: (0,) * a.ndim)
    return pl.pallas_call(
        _c0_kernel,
        grid=(grid,),
        in_specs=[nb(INPUT_DIM), full(w1f), full(w2f), full(wm0f)],
        out_specs=(nb(MUL), nb(SLICE), nb(SLICE), nb(SLICE)),
        out_shape=(
            jax.ShapeDtypeStruct((N, MUL), jnp.float32),
            jax.ShapeDtypeStruct((N, SLICE), jnp.float32),
            jax.ShapeDtypeStruct((N, SLICE), jnp.float32),
            jax.ShapeDtypeStruct((N, SLICE), jnp.float32),
        ),
    )(x, w1f, w2f, wm0f)


def _cl_kernel(nhw, rsin, rsout, h_ref, na_ref, a0_ref, a1_ref, a2_ref,
               ws_ref, wm_ref, h_out_ref, *hw_refs):
    agg = jnp.concatenate([
        a0_ref[0] + a0_ref[1],
        a1_ref[0] + a1_ref[1],
        a2_ref[0] + a2_ref[1],
    ], axis=1)
    sc = jnp.dot(h_ref[...] * na_ref[...], ws_ref[...],
                 preferred_element_type=jnp.float32) * rsin
    h = jax.nn.gelu(sc + agg)
    h_out_ref[...] = h
    hw = jnp.dot(h, wm_ref[...], preferred_element_type=jnp.float32) * rsout
    for s in range(nhw):
        hw_refs[s][...] = hw[:, SLICE * s:SLICE * (s + 1)]


def _cl(l, h_prev, node_attr, aggs, wsf, wmf):
    grid = N // NBLK
    din = h_prev.shape[1]
    nhw = NSLICES[l]
    nb = lambda d: pl.BlockSpec((NBLK, d), lambda i: (i, 0))
    ab = pl.BlockSpec((2, NBLK, SLICE), lambda i: (0, i, 0))
    full = lambda a: pl.BlockSpec(a.shape, lambda i: (0,) * a.ndim)
    return pl.pallas_call(
        functools.partial(_cl_kernel, nhw,
                          1.0 / math.sqrt(float(DIMS[l - 1])),
                          1.0 / math.sqrt(float(DIMS[l]))),
        grid=(grid,),
        in_specs=[nb(din), nb(1), ab, ab, ab, full(wsf), full(wmf)],
        out_specs=(nb(HPAD),) + tuple(nb(SLICE) for _ in range(nhw)),
        out_shape=(jax.ShapeDtypeStruct((N, HPAD), jnp.float32),)
        + tuple(jax.ShapeDtypeStruct((N, SLICE), jnp.float32)
                for _ in range(nhw)),
    )(h_prev, node_attr, aggs[0], aggs[1], aggs[2], wsf, wmf)


def _c4_kernel(h_ref, na_ref, a0_ref, ws_ref, out_ref):
    agg = a0_ref[0] + a0_ref[1]
    sc = jnp.dot(h_ref[...] * na_ref[...], ws_ref[...],
                 preferred_element_type=jnp.float32) \
        * (1.0 / math.sqrt(float(DIMS[NL - 1])))
    out_ref[...] = sc + agg[:, 0:DIMS[NL]]


def _c4(h_prev, node_attr, agg0, wsf):
    grid = N // NBLK
    nb = lambda d: pl.BlockSpec((NBLK, d), lambda i: (i, 0))
    ab = pl.BlockSpec((2, NBLK, SLICE), lambda i: (0, i, 0))
    full = lambda a: pl.BlockSpec(a.shape, lambda i: (0,) * a.ndim)
    return pl.pallas_call(
        _c4_kernel,
        grid=(grid,),
        in_specs=[nb(HPAD), nb(1), ab, full(wsf)],
        out_specs=nb(DIMS[NL]),
        out_shape=jax.ShapeDtypeStruct((N, DIMS[NL]), jnp.float32),
    )(h_prev, node_attr, agg0, wsf)


def kernel(x, pos, node_attr, edge_index, edge_attr, params):
    src = edge_index[0]
    dst = edge_index[1]

    pos16 = jnp.pad(pos, ((0, 0), (0, 13)))
    w1f = params['W1'] / math.sqrt(float(INPUT_DIM))
    w2f = params['W2'] / math.sqrt(float(MUL))

    f1s = jnp.concatenate([params[f'layer{l}']['F1'] for l in range(NL)],
                          axis=1)
    f2s = jnp.zeros((4 * MUL, 4 * 2 * MUL), jnp.float32)
    f3s = jnp.zeros((4 * 2 * MUL, 4 * HPAD), jnp.float32)
    for l in range(NL):
        f2s = f2s.at[MUL * l:MUL * (l + 1),
                     2 * MUL * l:2 * MUL * (l + 1)].set(
            params[f'layer{l}']['F2'])
        f3s = f3s.at[2 * MUL * l:2 * MUL * (l + 1),
                     HPAD * l:HPAD * l + DIMS[l + 1]].set(
            params[f'layer{l}']['F3'])
    wes = jnp.concatenate([
        jnp.pad(params[f'layer{l}']['We'],
                ((0, 0), (0, HPAD - DIMS[l + 1])))
        for l in range(NL)], axis=1)

    def wmf(l, cols):
        w = params[f'layer{l}']['Wm']
        return jnp.pad(w, ((0, (HPAD if l > 0 else INPUT_DIM) - DIMS[l]),
                           (0, cols - DIMS[l + 1])))

    def wsf(l, cols):
        w = params[f'layer{l}']['Ws']
        return jnp.pad(w, ((0, (HPAD if l > 0 else INPUT_DIM) - DIMS[l]),
                           (0, cols - DIMS[l + 1])))

    vec16 = _sc_edge_vec(pos16, src, dst)

    eaf16, emb16, p3d = _edge_dense(vec16, edge_attr, f1s, f2s, f3s, wes)
    p_flat = p3d.reshape(NPLANES * E, SLICE)

    h0, hw0, hw1, hw2 = _c0(x, w1f, w2f, wmf(0, HPAD))
    hws = [hw0, hw1, hw2]
    h = h0
    for l in range(NL - 1):
        aggs = _sc_message(l, src, dst, p_flat, hws)
        aggs = [a.reshape(2, N, SLICE) for a in aggs]
        cols = HPAD if l < NL - 2 else SLICE
        outs = _cl(l + 1, h, node_attr, aggs, wsf(l, HPAD), wmf(l + 1, cols))
        h = outs[0]
        hws = list(outs[1:])

    aggs = _sc_message(NL - 1, src, dst, p_flat, hws)
    out = _c4(h, node_attr, aggs[0].reshape(2, N, SLICE), wsf(NL - 1, DIMS[NL]))

    eaf = eaf16[:, 0:EDGE_ATTR_DIM]
    emb = emb16[:, 0:NB]
    batch = jnp.zeros((N,), dtype=jnp.int32)
    return (out, node_attr, src, dst, eaf, emb, batch)


# speedup vs baseline: 1.1544x; 1.1544x over previous
import functools
import math

import jax
import jax.numpy as jnp
from jax import lax
from jax.experimental import pallas as pl
from jax.experimental.pallas import tpu as pltpu
from jax.experimental.pallas import tpu_sc as plsc

N = 50000
E = 800000
INPUT_DIM = 16
MUL = 16
NB = 10
MAX_RADIUS = 3.5
NUM_NEI = 16
HID = 72
DIMS = [MUL, HID, HID, HID, 16]
NL = 4
EDGE_ATTR_DIM = 13

SLICE = 32
HPAD = 96
NSLICES = [3, 3, 3, 1]
POFF = [0, 3, 6, 9]
NPLANES = 10

CHUNK = 128
NCHUNKS = E // CHUNK
NWORK = 32
CPW = (NCHUNKS + NWORK - 1) // NWORK
RPT = N // 16
FCH = 125
NFL = RPT // FCH

EBLK = 1600
NBLK = 2000


def _sc_edge_vec(pos16, src, dst):
    mesh = plsc.VectorSubcoreMesh(core_axis_name="c", subcore_axis_name="s")

    @functools.partial(
        pl.kernel,
        out_type=jax.ShapeDtypeStruct((E, 16), jnp.float32),
        mesh=mesh,
        compiler_params=pltpu.CompilerParams(use_tc_tiling_on_sc=False),
        scratch_types=[
            pltpu.VMEM((CHUNK,), jnp.int32),
            pltpu.VMEM((CHUNK,), jnp.int32),
            pltpu.VMEM((CHUNK, 16), jnp.float32),
            pltpu.VMEM((CHUNK, 16), jnp.float32),
            pltpu.SemaphoreType.DMA,
            pltpu.SemaphoreType.DMA,
        ],
    )
    def k(pos_ref, src_ref, dst_ref, vec_ref, sidx, didx, pb, qb, sem1, sem2):
        wid = lax.axis_index("s") * 2 + lax.axis_index("c")

        def chunk(i, carry):
            c = wid + NWORK * i

            @pl.when(c < NCHUNKS)
            def _():
                base = c * CHUNK
                pltpu.sync_copy(src_ref.at[pl.ds(base, CHUNK)], sidx)
                pltpu.sync_copy(dst_ref.at[pl.ds(base, CHUNK)], didx)
                cp1 = pltpu.async_copy(pos_ref.at[sidx], pb, sem1)
                cp2 = pltpu.async_copy(pos_ref.at[didx], qb, sem2)
                cp1.wait()
                cp2.wait()

                def sub(j, cc):
                    pb[j, pl.ds(0, 16)] = pb[j, pl.ds(0, 16)] - qb[j, pl.ds(0, 16)]
                    return cc

                lax.fori_loop(0, CHUNK, sub, 0)
                pltpu.sync_copy(pb, vec_ref.at[pl.ds(base, CHUNK)])

            return carry

        lax.fori_loop(0, CPW, chunk, 0)

    return k(pos16, src, dst)


G = 128
SUB = G // CHUNK
NGR = E // G
IPW = (NGR + NWORK - 1) // NWORK
OUTER = IPW // 2


def _sc_message(l, src, dst, ps, hws):
    S = NSLICES[l]
    mesh = plsc.VectorSubcoreMesh(core_axis_name="c", subcore_axis_name="s")
    scratch = [
        [pltpu.VMEM((G,), jnp.int32) for _ in range(2)],
        [pltpu.VMEM((SUB, CHUNK), jnp.int32) for _ in range(2)],
        [pltpu.VMEM((SUB, CHUNK), jnp.int32) for _ in range(2)],
        [pltpu.VMEM((G, SLICE), jnp.float32) for _ in range(2)],
        [pltpu.VMEM((G, SLICE), jnp.float32) for _ in range(2)],
        [pltpu.VMEM((G, SLICE), jnp.float32) for _ in range(2)],
        pltpu.VMEM((FCH, SLICE), jnp.float32),
        pltpu.VMEM_SHARED((N, SLICE), jnp.float32),
        [pltpu.SemaphoreType.DMA for _ in range(2)],
        [pltpu.SemaphoreType.DMA for _ in range(2)],
        pltpu.SemaphoreType.DMA,
    ]

    @functools.partial(
        pl.kernel,
        out_type=tuple(jax.ShapeDtypeStruct((2 * N, SLICE), jnp.float32)
                       for _ in range(S)),
        mesh=mesh,
        compiler_params=pltpu.CompilerParams(use_tc_tiling_on_sc=False),
        scratch_types=scratch,
    )
    def k(src_ref, dst_ref, *rest):
        p_refs = rest[:S]
        hw_refs = rest[S:2 * S]
        agg_refs = rest[2 * S:3 * S]
        (sidx, didx, sdix, hb, pb, mb, zb, acc, gsem, ssem, fsem) = rest[3 * S:]
        cid = lax.axis_index("c")
        tid = lax.axis_index("s")
        wid = tid * 2 + cid

        def z(j, c):
            zb[j, pl.ds(0, 16)] = jnp.zeros((16,), jnp.float32)
            zb[j, pl.ds(16, 16)] = jnp.zeros((16,), jnp.float32)
            return c

        lax.fori_loop(0, FCH, z, 0)

        for s in range(S):
            hw_ref = hw_refs[s]
            p_ref = p_refs[s]

            for f0 in range(0, NFL, 5):
                zcps = [pltpu.async_copy(
                    zb, acc.at[pl.ds(tid * RPT + f * FCH, FCH)], fsem)
                    for f in range(f0, min(NFL, f0 + 5))]
                for cp in zcps:
                    cp.wait()
            plsc.subcore_barrier()

            def issue(i, b):
                c = wid + NWORK * i

                @pl.when(c < NGR)
                def _():
                    base = c * G
                    pltpu.sync_copy(src_ref.at[pl.ds(base, G)], sidx[b])
                    for j in range(SUB):
                        pltpu.sync_copy(
                            dst_ref.at[pl.ds(base + CHUNK * j, CHUNK)],
                            didx[b].at[j])
                    for j in range(SUB):
                        pltpu.async_copy(
                            hw_ref.at[sidx[b].at[pl.ds(CHUNK * j, CHUNK)]],
                            hb[b].at[pl.ds(CHUNK * j, CHUNK)], gsem[b])
                    pltpu.async_copy(p_ref.at[pl.ds(base, G)],
                                     pb[b], gsem[b])

            def process(i, i2, b):
                c = wid + NWORK * i

                @pl.when(c < NGR)
                def _():
                    for j in range(SUB):
                        pltpu.make_async_copy(
                            hw_ref.at[sidx[b].at[pl.ds(CHUNK * j, CHUNK)]],
                            hb[b].at[pl.ds(CHUNK * j, CHUNK)], gsem[b]).wait()
                    pltpu.make_async_copy(p_ref.at[pl.ds(0, G)],
                                          pb[b], gsem[b]).wait()

                    @pl.when(i2 >= 1)
                    def _():
                        for j in range(SUB):
                            pltpu.make_async_copy(
                                mb[b].at[pl.ds(CHUNK * j, CHUNK)],
                                acc.at[sdix[b].at[j]], ssem[b]).wait()

                    for j in range(SUB):
                        for t in range(CHUNK // 16):
                            sdix[b][j, pl.ds(16 * t, 16)] = \
                                didx[b][j, pl.ds(16 * t, 16)]

                    def mul(jj, cc):
                        for r in range(4):
                            row = jj * 4 + r
                            mb[b][row, pl.ds(0, 16)] = \
                                pb[b][row, pl.ds(0, 16)] * hb[b][row, pl.ds(0, 16)]
                            mb[b][row, pl.ds(16, 16)] = \
                                pb[b][row, pl.ds(16, 16)] * hb[b][row, pl.ds(16, 16)]
                        return cc

                    lax.fori_loop(0, G // 4, mul, 0)

                    for j in range(SUB):
                        pltpu.async_copy(
                            mb[b].at[pl.ds(CHUNK * j, CHUNK)],
                            acc.at[sdix[b].at[j]], ssem[b], add=True)

            issue(0, 0)
            issue(1, 1)

            def outer(i2, carry):
                for b in range(2):
                    i = i2 * 2 + b
                    process(i, i2, b)
                    issue(i + 2, b)
                return carry

            lax.fori_loop(0, OUTER, outer, 0)

            for b in range(2):
                for j in range(SUB):
                    pltpu.make_async_copy(
                        mb[b].at[pl.ds(CHUNK * j, CHUNK)],
                        acc.at[sdix[b].at[j]], ssem[b]).wait()

            plsc.subcore_barrier()

            for f0 in range(0, NFL, 5):
                fcps = [pltpu.async_copy(
                    acc.at[pl.ds(tid * RPT + f * FCH, FCH)],
                    agg_refs[s].at[pl.ds(cid * N + tid * RPT + f * FCH, FCH)],
                    fsem) for f in range(f0, min(NFL, f0 + 5))]
                for cp in fcps:
                    cp.wait()
            plsc.subcore_barrier()

    return k(src, dst, *ps, *hws)


def _edge_dense_kernel(vec_ref, ea_ref, f1_ref, f2_ref, f3_ref, we_ref,
                       eaf_ref, emb_ref, *p_refs):
    vec = vec_ref[...][:, 0:3]
    r2 = jnp.sum(vec * vec, axis=1, keepdims=True) + 1e-12
    r = jnp.sqrt(r2)
    u = vec / r
    x, y, z = u[:, 0:1], u[:, 1:2], u[:, 2:3]
    s3 = math.sqrt(3.0)
    s15 = math.sqrt(15.0)
    s5h = math.sqrt(5.0) / 2.0
    zero = jnp.zeros_like(x)
    eaf16 = jnp.concatenate([
        ea_ref[...],
        jnp.ones_like(x), s3 * x, s3 * y, s3 * z,
        s15 * x * y, s15 * y * z, s5h * (3.0 * z * z - 1.0),
        s15 * x * z, (s15 / 2.0) * (x * x - y * y),
        zero, zero, zero,
    ], axis=1)
    eaf_ref[...] = eaf16
    eaf = eaf16[:, 0:EDGE_ATTR_DIM]

    step = MAX_RADIUS / (NB + 1)
    values = (lax.broadcasted_iota(jnp.int32, (1, NB), 1)
              .astype(jnp.float32) + 1.0) * step
    diff = (r - values) / step
    theta = jnp.reshape(r, (1, r.shape[0])) * (jnp.pi / 2.0 / step)
    ct = jnp.reshape(jnp.cos(theta), (r.shape[0], 1))
    st = jnp.reshape(jnp.sin(theta), (r.shape[0], 1))
    m4 = jnp.remainder(lax.broadcasted_iota(jnp.int32, (1, NB), 1) + 1, 4)
    ck = jnp.where(m4 == 0, 1.0, 0.0) + jnp.where(m4 == 2, -1.0, 0.0)
    sk = jnp.where(m4 == 1, 1.0, 0.0) + jnp.where(m4 == 3, -1.0, 0.0)
    emb = (ct * ck + st * sk) * (diff > -1.0) * (diff < 1.0)
    emb = emb * (NB ** 0.5)
    emb_ref[...] = jnp.concatenate([emb, jnp.zeros_like(emb[:, 0:6])], axis=1)

    e1 = jax.nn.gelu(jnp.dot(emb, f1_ref[...],
                             preferred_element_type=jnp.float32)
                     / math.sqrt(float(NB)))
    e2 = jax.nn.gelu(jnp.dot(e1, f2_ref[...],
                             preferred_element_type=jnp.float32)
                     / math.sqrt(float(MUL)))
    w_all = jnp.dot(e2, f3_ref[...], preferred_element_type=jnp.float32) \
        / math.sqrt(float(2 * MUL))
    a_all = jnp.dot(eaf, we_ref[...], preferred_element_type=jnp.float32) \
        / math.sqrt(float(EDGE_ATTR_DIM))
    p_all = w_all * a_all * (1.0 / math.sqrt(float(NUM_NEI)))
    for l in range(NL):
        for s in range(NSLICES[l]):
            c0 = HPAD * l + SLICE * s
            p_refs[POFF[l] + s][...] = p_all[:, c0:c0 + SLICE]


def _edge_dense(vec16, ea, f1s, f2s, f3s, wes):
    grid = E // EBLK
    eb = lambda d: pl.BlockSpec((EBLK, d), lambda i: (i, 0))
    full = lambda a: pl.BlockSpec(a.shape, lambda i: (0,) * a.ndim)
    return pl.pallas_call(
        _edge_dense_kernel,
        grid=(grid,),
        in_specs=[eb(16), eb(4), full(f1s), full(f2s), full(f3s), full(wes)],
        out_specs=(eb(16), eb(16)) + tuple(eb(SLICE) for _ in range(NPLANES)),
        out_shape=(
            jax.ShapeDtypeStruct((E, 16), jnp.float32),
            jax.ShapeDtypeStruct((E, 16), jnp.float32),
        ) + tuple(jax.ShapeDtypeStruct((E, SLICE), jnp.float32)
                  for _ in range(NPLANES)),
    )(vec16, ea, f1s, f2s, f3s, wes)


def _c0_kernel(x_ref, w1_ref, w2_ref, wm_ref, h0_ref, hw0_ref, hw1_ref, hw2_ref):
    h0 = jax.nn.gelu(jnp.dot(x_ref[...], w1_ref[...],
                             preferred_element_type=jnp.float32))
    h0 = jnp.dot(h0, w2_ref[...], preferred_element_type=jnp.float32)
    h0_ref[...] = h0
    hw = jnp.dot(h0, wm_ref[...], preferred_element_type=jnp.float32) * 0.25
    hw0_ref[...] = hw[:, 0:SLICE]
    hw1_ref[...] = hw[:, SLICE:2 * SLICE]
    hw2_ref[...] = hw[:, 2 * SLICE:3 * SLICE]


def _c0(x, w1f, w2f, wm0f):
    grid = N // NBLK
    nb = lambda d: pl.BlockSpec((NBLK, d), lambda i: (i, 0))
    full = lambda a: pl.BlockSpec(a.shape, lambda i: (0,) * a.ndim)
    return pl.pallas_call(
        _c0_kernel,
        grid=(grid,),
        in_specs=[nb(INPUT_DIM), full(w1f), full(w2f), full(wm0f)],
        out_specs=(nb(MUL), nb(SLICE), nb(SLICE), nb(SLICE)),
        out_shape=(
            jax.ShapeDtypeStruct((N, MUL), jnp.float32),
            jax.ShapeDtypeStruct((N, SLICE), jnp.float32),
            jax.ShapeDtypeStruct((N, SLICE), jnp.float32),
            jax.ShapeDtypeStruct((N, SLICE), jnp.float32),
        ),
    )(x, w1f, w2f, wm0f)


def _cl_kernel(nhw, rsin, rsout, h_ref, na_ref, a0_ref, a1_ref, a2_ref,
               ws_ref, wm_ref, h_out_ref, *hw_refs):
    agg = jnp.concatenate([
        a0_ref[0] + a0_ref[1],
        a1_ref[0] + a1_ref[1],
        a2_ref[0] + a2_ref[1],
    ], axis=1)
    sc = jnp.dot(h_ref[...] * na_ref[...], ws_ref[...],
                 preferred_element_type=jnp.float32) * rsin
    h = jax.nn.gelu(sc + agg)
    h_out_ref[...] = h
    hw = jnp.dot(h, wm_ref[...], preferred_element_type=jnp.float32) * rsout
    for s in range(nhw):
        hw_refs[s][...] = hw[:, SLICE * s:SLICE * (s + 1)]


def _cl(l, h_prev, node_attr, aggs, wsf, wmf):
    grid = N // NBLK
    din = h_prev.shape[1]
    nhw = NSLICES[l]
    nb = lambda d: pl.BlockSpec((NBLK, d), lambda i: (i, 0))
    ab = pl.BlockSpec((2, NBLK, SLICE), lambda i: (0, i, 0))
    full = lambda a: pl.BlockSpec(a.shape, lambda i: (0,) * a.ndim)
    return pl.pallas_call(
        functools.partial(_cl_kernel, nhw,
                          1.0 / math.sqrt(float(DIMS[l - 1])),
                          1.0 / math.sqrt(float(DIMS[l]))),
        grid=(grid,),
        in_specs=[nb(din), nb(1), ab, ab, ab, full(wsf), full(wmf)],
        out_specs=(nb(HPAD),) + tuple(nb(SLICE) for _ in range(nhw)),
        out_shape=(jax.ShapeDtypeStruct((N, HPAD), jnp.float32),)
        + tuple(jax.ShapeDtypeStruct((N, SLICE), jnp.float32)
                for _ in range(nhw)),
    )(h_prev, node_attr, aggs[0], aggs[1], aggs[2], wsf, wmf)


def _c4_kernel(h_ref, na_ref, a0_ref, ws_ref, out_ref):
    agg = a0_ref[0] + a0_ref[1]
    sc = jnp.dot(h_ref[...] * na_ref[...], ws_ref[...],
                 preferred_element_type=jnp.float32) \
        * (1.0 / math.sqrt(float(DIMS[NL - 1])))
    out_ref[...] = sc + agg[:, 0:DIMS[NL]]


def _c4(h_prev, node_attr, agg0, wsf):
    grid = N // NBLK
    nb = lambda d: pl.BlockSpec((NBLK, d), lambda i: (i, 0))
    ab = pl.BlockSpec((2, NBLK, SLICE), lambda i: (0, i, 0))
    full = lambda a: pl.BlockSpec(a.shape, lambda i: (0,) * a.ndim)
    return pl.pallas_call(
        _c4_kernel,
        grid=(grid,),
        in_specs=[nb(HPAD), nb(1), ab, full(wsf)],
        out_specs=nb(DIMS[NL]),
        out_shape=jax.ShapeDtypeStruct((N, DIMS[NL]), jnp.float32),
    )(h_prev, node_attr, agg0, wsf)


def kernel(x, pos, node_attr, edge_index, edge_attr, params):
    src = edge_index[0]
    dst = edge_index[1]

    pos16 = jnp.pad(pos, ((0, 0), (0, 13)))
    w1f = params['W1'] / math.sqrt(float(INPUT_DIM))
    w2f = params['W2'] / math.sqrt(float(MUL))

    f1s = jnp.concatenate([params[f'layer{l}']['F1'] for l in range(NL)],
                          axis=1)
    f2s = jnp.zeros((4 * MUL, 4 * 2 * MUL), jnp.float32)
    f3s = jnp.zeros((4 * 2 * MUL, 4 * HPAD), jnp.float32)
    for l in range(NL):
        f2s = f2s.at[MUL * l:MUL * (l + 1),
                     2 * MUL * l:2 * MUL * (l + 1)].set(
            params[f'layer{l}']['F2'])
        f3s = f3s.at[2 * MUL * l:2 * MUL * (l + 1),
                     HPAD * l:HPAD * l + DIMS[l + 1]].set(
            params[f'layer{l}']['F3'])
    wes = jnp.concatenate([
        jnp.pad(params[f'layer{l}']['We'],
                ((0, 0), (0, HPAD - DIMS[l + 1])))
        for l in range(NL)], axis=1)

    def wmf(l, cols):
        w = params[f'layer{l}']['Wm']
        return jnp.pad(w, ((0, (HPAD if l > 0 else INPUT_DIM) - DIMS[l]),
                           (0, cols - DIMS[l + 1])))

    def wsf(l, cols):
        w = params[f'layer{l}']['Ws']
        return jnp.pad(w, ((0, (HPAD if l > 0 else INPUT_DIM) - DIMS[l]),
                           (0, cols - DIMS[l + 1])))

    vec16 = _sc_edge_vec(pos16, src, dst)

    edouts = _edge_dense(vec16, edge_attr, f1s, f2s, f3s, wes)
    eaf16, emb16 = edouts[0], edouts[1]
    planes = edouts[2:]

    h0, hw0, hw1, hw2 = _c0(x, w1f, w2f, wmf(0, HPAD))
    hws = [hw0, hw1, hw2]
    h = h0
    for l in range(NL - 1):
        aggs = _sc_message(l, src, dst,
                           planes[POFF[l]:POFF[l] + NSLICES[l]], hws)
        aggs = [a.reshape(2, N, SLICE) for a in aggs]
        cols = HPAD if l < NL - 2 else SLICE
        outs = _cl(l + 1, h, node_attr, aggs, wsf(l, HPAD), wmf(l + 1, cols))
        h = outs[0]
        hws = list(outs[1:])

    aggs = _sc_message(NL - 1, src, dst,
                       planes[POFF[NL - 1]:POFF[NL - 1] + 1], hws)
    out = _c4(h, node_attr, aggs[0].reshape(2, N, SLICE), wsf(NL - 1, DIMS[NL]))

    eaf = eaf16[:, 0:EDGE_ATTR_DIM]
    emb = emb16[:, 0:NB]
    batch = jnp.zeros((N,), dtype=jnp.int32)
    return (out, node_attr, src, dst, eaf, emb, batch)


# speedup vs baseline: 1.3396x; 1.1604x over previous
import functools
import math

import jax
import jax.numpy as jnp
from jax import lax
from jax.experimental import pallas as pl
from jax.experimental.pallas import tpu as pltpu
from jax.experimental.pallas import tpu_sc as plsc

N = 50000
E = 800000
INPUT_DIM = 16
MUL = 16
NB = 10
MAX_RADIUS = 3.5
NUM_NEI = 16
HID = 72
DIMS = [MUL, HID, HID, HID, 16]
NL = 4
EDGE_ATTR_DIM = 13

SLICE = 32
HPAD = 96
NSLICES = [3, 3, 3, 1]
POFF = [0, 3, 6, 9]
NPLANES = 10

CHUNK = 128
NCHUNKS = E // CHUNK
NWORK = 32
CPW = (NCHUNKS + NWORK - 1) // NWORK
RPT = N // 16
FCH = 125
NFL = RPT // FCH

EBLK = 1600
NBLK = 2000


def _sc_edge_vec(pos16, src, dst):
    mesh = plsc.VectorSubcoreMesh(core_axis_name="c", subcore_axis_name="s")

    @functools.partial(
        pl.kernel,
        out_type=jax.ShapeDtypeStruct((E, 16), jnp.float32),
        mesh=mesh,
        compiler_params=pltpu.CompilerParams(use_tc_tiling_on_sc=False),
        scratch_types=[
            pltpu.VMEM((CHUNK,), jnp.int32),
            pltpu.VMEM((CHUNK,), jnp.int32),
            pltpu.VMEM((CHUNK, 16), jnp.float32),
            pltpu.VMEM((CHUNK, 16), jnp.float32),
            pltpu.SemaphoreType.DMA,
            pltpu.SemaphoreType.DMA,
        ],
    )
    def k(pos_ref, src_ref, dst_ref, vec_ref, sidx, didx, pb, qb, sem1, sem2):
        wid = lax.axis_index("s") * 2 + lax.axis_index("c")

        def chunk(i, carry):
            c = wid + NWORK * i

            @pl.when(c < NCHUNKS)
            def _():
                base = c * CHUNK
                pltpu.sync_copy(src_ref.at[pl.ds(base, CHUNK)], sidx)
                pltpu.sync_copy(dst_ref.at[pl.ds(base, CHUNK)], didx)
                cp1 = pltpu.async_copy(pos_ref.at[sidx], pb, sem1)
                cp2 = pltpu.async_copy(pos_ref.at[didx], qb, sem2)
                cp1.wait()
                cp2.wait()

                def sub(j, cc):
                    pb[j, pl.ds(0, 16)] = pb[j, pl.ds(0, 16)] - qb[j, pl.ds(0, 16)]
                    return cc

                lax.fori_loop(0, CHUNK, sub, 0)
                pltpu.sync_copy(pb, vec_ref.at[pl.ds(base, CHUNK)])

            return carry

        lax.fori_loop(0, CPW, chunk, 0)

    return k(pos16, src, dst)


G = 128
SUB = G // CHUNK
NGR = E // G
IPW = (NGR + NWORK - 1) // NWORK
OUTER = IPW // 2


def _sc_message(l, src, dst, ps, hws):
    S = NSLICES[l]
    mesh = plsc.VectorSubcoreMesh(core_axis_name="c", subcore_axis_name="s")
    scratch = [
        [pltpu.VMEM((G,), jnp.int32) for _ in range(2)],
        [pltpu.VMEM((SUB, CHUNK), jnp.int32) for _ in range(2)],
        [pltpu.VMEM((SUB, CHUNK), jnp.int32) for _ in range(2)],
        [pltpu.VMEM((G, SLICE), jnp.float32) for _ in range(2)],
        [pltpu.VMEM((G, SLICE), jnp.float32) for _ in range(2)],
        [pltpu.VMEM((G, SLICE), jnp.float32) for _ in range(2)],
        pltpu.VMEM((FCH, SLICE), jnp.float32),
        pltpu.VMEM_SHARED((N, SLICE), jnp.float32),
        [pltpu.SemaphoreType.DMA for _ in range(2)],
        [pltpu.SemaphoreType.DMA for _ in range(2)],
        pltpu.SemaphoreType.DMA,
    ]

    @functools.partial(
        pl.kernel,
        out_type=tuple(jax.ShapeDtypeStruct((2 * N, SLICE), jnp.float32)
                       for _ in range(S)),
        mesh=mesh,
        compiler_params=pltpu.CompilerParams(use_tc_tiling_on_sc=False),
        scratch_types=scratch,
    )
    def k(src_ref, dst_ref, *rest):
        p_refs = rest[:S]
        hw_refs = rest[S:2 * S]
        agg_refs = rest[2 * S:3 * S]
        (sidx, didx, sdix, hb, pb, mb, zb, acc, gsem, ssem, fsem) = rest[3 * S:]
        cid = lax.axis_index("c")
        tid = lax.axis_index("s")
        wid = tid * 2 + cid

        def z(j, c):
            zb[j, pl.ds(0, 16)] = jnp.zeros((16,), jnp.float32)
            zb[j, pl.ds(16, 16)] = jnp.zeros((16,), jnp.float32)
            return c

        lax.fori_loop(0, FCH, z, 0)

        for s in range(S):
            hw_ref = hw_refs[s]
            p_ref = p_refs[s]

            for f0 in range(0, NFL, 5):
                zcps = [pltpu.async_copy(
                    zb, acc.at[pl.ds(tid * RPT + f * FCH, FCH)], fsem)
                    for f in range(f0, min(NFL, f0 + 5))]
                for cp in zcps:
                    cp.wait()
            plsc.subcore_barrier()

            def issue(i, b):
                c = wid + NWORK * i

                @pl.when(c < NGR)
                def _():
                    base = c * G
                    pltpu.sync_copy(src_ref.at[pl.ds(base, G)], sidx[b])
                    for j in range(SUB):
                        pltpu.sync_copy(
                            dst_ref.at[pl.ds(base + CHUNK * j, CHUNK)],
                            didx[b].at[j])
                    for j in range(SUB):
                        pltpu.async_copy(
                            hw_ref.at[sidx[b].at[pl.ds(CHUNK * j, CHUNK)]],
                            hb[b].at[pl.ds(CHUNK * j, CHUNK)], gsem[b])
                    pltpu.async_copy(p_ref.at[pl.ds(base, G)],
                                     pb[b], gsem[b])

            def process(i, i2, b):
                c = wid + NWORK * i

                @pl.when(c < NGR)
                def _():
                    for j in range(SUB):
                        pltpu.make_async_copy(
                            hw_ref.at[sidx[b].at[pl.ds(CHUNK * j, CHUNK)]],
                            hb[b].at[pl.ds(CHUNK * j, CHUNK)], gsem[b]).wait()
                    pltpu.make_async_copy(p_ref.at[pl.ds(0, G)],
                                          pb[b], gsem[b]).wait()

                    @pl.when(i2 >= 1)
                    def _():
                        for j in range(SUB):
                            pltpu.make_async_copy(
                                mb[b].at[pl.ds(CHUNK * j, CHUNK)],
                                acc.at[sdix[b].at[j]], ssem[b]).wait()

                    for j in range(SUB):
                        for t in range(CHUNK // 16):
                            sdix[b][j, pl.ds(16 * t, 16)] = \
                                didx[b][j, pl.ds(16 * t, 16)]

                    def mul(jj, cc):
                        for r in range(4):
                            row = jj * 4 + r
                            mb[b][row, pl.ds(0, 16)] = \
                                pb[b][row, pl.ds(0, 16)] * hb[b][row, pl.ds(0, 16)]
                            mb[b][row, pl.ds(16, 16)] = \
                                pb[b][row, pl.ds(16, 16)] * hb[b][row, pl.ds(16, 16)]
                        return cc

                    lax.fori_loop(0, G // 4, mul, 0)

                    for j in range(SUB):
                        pltpu.async_copy(
                            mb[b].at[pl.ds(CHUNK * j, CHUNK)],
                            acc.at[sdix[b].at[j]], ssem[b], add=True)

            issue(0, 0)
            issue(1, 1)

            def outer(i2, carry):
                for b in range(2):
                    i = i2 * 2 + b
                    process(i, i2, b)
                    issue(i + 2, b)
                return carry

            lax.fori_loop(0, OUTER, outer, 0)

            for b in range(2):
                for j in range(SUB):
                    pltpu.make_async_copy(
                        mb[b].at[pl.ds(CHUNK * j, CHUNK)],
                        acc.at[sdix[b].at[j]], ssem[b]).wait()

            plsc.subcore_barrier()

            for f0 in range(0, NFL, 5):
                fcps = [pltpu.async_copy(
                    acc.at[pl.ds(tid * RPT + f * FCH, FCH)],
                    agg_refs[s].at[pl.ds(cid * N + tid * RPT + f * FCH, FCH)],
                    fsem) for f in range(f0, min(NFL, f0 + 5))]
                for cp in fcps:
                    cp.wait()
            plsc.subcore_barrier()

    return k(src, dst, *ps, *hws)


def _edge_dense_kernel(vec_ref, ea_ref, f1_ref, f2_ref, f3_ref, we_ref,
                       eaf_ref, emb_ref, *p_refs):
    vec = vec_ref[...][:, 0:3]
    r2 = jnp.sum(vec * vec, axis=1, keepdims=True) + 1e-12
    r = jnp.sqrt(r2)
    u = vec / r
    x, y, z = u[:, 0:1], u[:, 1:2], u[:, 2:3]
    s3 = math.sqrt(3.0)
    s15 = math.sqrt(15.0)
    s5h = math.sqrt(5.0) / 2.0
    zero = jnp.zeros_like(x)
    eaf16 = jnp.concatenate([
        ea_ref[...],
        jnp.ones_like(x), s3 * x, s3 * y, s3 * z,
        s15 * x * y, s15 * y * z, s5h * (3.0 * z * z - 1.0),
        s15 * x * z, (s15 / 2.0) * (x * x - y * y),
        zero, zero, zero,
    ], axis=1)
    eaf_ref[...] = eaf16
    eaf = eaf16[:, 0:EDGE_ATTR_DIM]

    step = MAX_RADIUS / (NB + 1)
    values = (lax.broadcasted_iota(jnp.int32, (1, NB), 1)
              .astype(jnp.float32) + 1.0) * step
    diff = (r - values) / step
    u2 = diff * diff
    pc = ((((0.0008581625461597372 * u2 - 0.02081057153075854) * u2
            + 0.25365070978091486) * u2 - 1.233698207748837) * u2
          + 0.9999999532569678)
    emb = pc * (diff > -1.0) * (diff < 1.0)
    emb = emb * (NB ** 0.5)
    emb_ref[...] = jnp.concatenate([emb, jnp.zeros_like(emb[:, 0:6])], axis=1)

    e1 = jax.nn.gelu(jnp.dot(emb, f1_ref[...],
                             preferred_element_type=jnp.float32)
                     / math.sqrt(float(NB)))
    e2 = jax.nn.gelu(jnp.dot(e1, f2_ref[...],
                             preferred_element_type=jnp.float32)
                     / math.sqrt(float(MUL)))
    w_all = jnp.dot(e2, f3_ref[...], preferred_element_type=jnp.float32) \
        / math.sqrt(float(2 * MUL))
    a_all = jnp.dot(eaf, we_ref[...], preferred_element_type=jnp.float32) \
        / math.sqrt(float(EDGE_ATTR_DIM))
    p_all = w_all * a_all * (1.0 / math.sqrt(float(NUM_NEI)))
    for l in range(NL):
        for s in range(NSLICES[l]):
            c0 = HPAD * l + SLICE * s
            p_refs[POFF[l] + s][...] = p_all[:, c0:c0 + SLICE]


def _edge_dense(vec16, ea, f1s, f2s, f3s, wes):
    grid = E // EBLK
    eb = lambda d: pl.BlockSpec((EBLK, d), lambda i: (i, 0))
    full = lambda a: pl.BlockSpec(a.shape, lambda i: (0,) * a.ndim)
    return pl.pallas_call(
        _edge_dense_kernel,
        grid=(grid,),
        in_specs=[eb(16), eb(4), full(f1s), full(f2s), full(f3s), full(wes)],
        out_specs=(eb(16), eb(16)) + tuple(eb(SLICE) for _ in range(NPLANES)),
        out_shape=(
            jax.ShapeDtypeStruct((E, 16), jnp.float32),
            jax.ShapeDtypeStruct((E, 16), jnp.float32),
        ) + tuple(jax.ShapeDtypeStruct((E, SLICE), jnp.float32)
                  for _ in range(NPLANES)),
    )(vec16, ea, f1s, f2s, f3s, wes)


def _c0_kernel(x_ref, w1_ref, w2_ref, wm_ref, h0_ref, hw0_ref, hw1_ref, hw2_ref):
    h0 = jax.nn.gelu(jnp.dot(x_ref[...], w1_ref[...],
                             preferred_element_type=jnp.float32))
    h0 = jnp.dot(h0, w2_ref[...], preferred_element_type=jnp.float32)
    h0_ref[...] = h0
    hw = jnp.dot(h0, wm_ref[...], preferred_element_type=jnp.float32) * 0.25
    hw0_ref[...] = hw[:, 0:SLICE]
    hw1_ref[...] = hw[:, SLICE:2 * SLICE]
    hw2_ref[...] = hw[:, 2 * SLICE:3 * SLICE]


def _c0(x, w1f, w2f, wm0f):
    grid = N // NBLK
    nb = lambda d: pl.BlockSpec((NBLK, d), lambda i: (i, 0))
    full = lambda a: pl.BlockSpec(a.shape, lambda i: (0,) * a.ndim)
    return pl.pallas_call(
        _c0_kernel,
        grid=(grid,),
        in_specs=[nb(INPUT_DIM), full(w1f), full(w2f), full(wm0f)],
        out_specs=(nb(MUL), nb(SLICE), nb(SLICE), nb(SLICE)),
        out_shape=(
            jax.ShapeDtypeStruct((N, MUL), jnp.float32),
            jax.ShapeDtypeStruct((N, SLICE), jnp.float32),
            jax.ShapeDtypeStruct((N, SLICE), jnp.float32),
            jax.ShapeDtypeStruct((N, SLICE), jnp.float32),
        ),
    )(x, w1f, w2f, wm0f)


def _cl_kernel(nhw, rsin, rsout, h_ref, na_ref, a0_ref, a1_ref, a2_ref,
               ws_ref, wm_ref, h_out_ref, *hw_refs):
    agg = jnp.concatenate([
        a0_ref[0] + a0_ref[1],
        a1_ref[0] + a1_ref[1],
        a2_ref[0] + a2_ref[1],
    ], axis=1)
    sc = jnp.dot(h_ref[...] * na_ref[...], ws_ref[...],
                 preferred_element_type=jnp.float32) * rsin
    h = jax.nn.gelu(sc + agg)
    h_out_ref[...] = h
    hw = jnp.dot(h, wm_ref[...], preferred_element_type=jnp.float32) * rsout
    for s in range(nhw):
        hw_refs[s][...] = hw[:, SLICE * s:SLICE * (s + 1)]


def _cl(l, h_prev, node_attr, aggs, wsf, wmf):
    grid = N // NBLK
    din = h_prev.shape[1]
    nhw = NSLICES[l]
    nb = lambda d: pl.BlockSpec((NBLK, d), lambda i: (i, 0))
    ab = pl.BlockSpec((2, NBLK, SLICE), lambda i: (0, i, 0))
    full = lambda a: pl.BlockSpec(a.shape, lambda i: (0,) * a.ndim)
    return pl.pallas_call(
        functools.partial(_cl_kernel, nhw,
                          1.0 / math.sqrt(float(DIMS[l - 1])),
                          1.0 / math.sqrt(float(DIMS[l]))),
        grid=(grid,),
        in_specs=[nb(din), nb(1), ab, ab, ab, full(wsf), full(wmf)],
        out_specs=(nb(HPAD),) + tuple(nb(SLICE) for _ in range(nhw)),
        out_shape=(jax.ShapeDtypeStruct((N, HPAD), jnp.float32),)
        + tuple(jax.ShapeDtypeStruct((N, SLICE), jnp.float32)
                for _ in range(nhw)),
    )(h_prev, node_attr, aggs[0], aggs[1], aggs[2], wsf, wmf)


def _c4_kernel(h_ref, na_ref, a0_ref, ws_ref, out_ref):
    agg = a0_ref[0] + a0_ref[1]
    sc = jnp.dot(h_ref[...] * na_ref[...], ws_ref[...],
                 preferred_element_type=jnp.float32) \
        * (1.0 / math.sqrt(float(DIMS[NL - 1])))
    out_ref[...] = sc + agg[:, 0:DIMS[NL]]


def _c4(h_prev, node_attr, agg0, wsf):
    grid = N // NBLK
    nb = lambda d: pl.BlockSpec((NBLK, d), lambda i: (i, 0))
    ab = pl.BlockSpec((2, NBLK, SLICE), lambda i: (0, i, 0))
    full = lambda a: pl.BlockSpec(a.shape, lambda i: (0,) * a.ndim)
    return pl.pallas_call(
        _c4_kernel,
        grid=(grid,),
        in_specs=[nb(HPAD), nb(1), ab, full(wsf)],
        out_specs=nb(DIMS[NL]),
        out_shape=jax.ShapeDtypeStruct((N, DIMS[NL]), jnp.float32),
    )(h_prev, node_attr, agg0, wsf)


def kernel(x, pos, node_attr, edge_index, edge_attr, params):
    src = edge_index[0]
    dst = edge_index[1]

    pos16 = jnp.pad(pos, ((0, 0), (0, 13)))
    w1f = params['W1'] / math.sqrt(float(INPUT_DIM))
    w2f = params['W2'] / math.sqrt(float(MUL))

    f1s = jnp.concatenate([params[f'layer{l}']['F1'] for l in range(NL)],
                          axis=1)
    f2s = jnp.zeros((4 * MUL, 4 * 2 * MUL), jnp.float32)
    f3s = jnp.zeros((4 * 2 * MUL, 4 * HPAD), jnp.float32)
    for l in range(NL):
        f2s = f2s.at[MUL * l:MUL * (l + 1),
                     2 * MUL * l:2 * MUL * (l + 1)].set(
            params[f'layer{l}']['F2'])
        f3s = f3s.at[2 * MUL * l:2 * MUL * (l + 1),
                     HPAD * l:HPAD * l + DIMS[l + 1]].set(
            params[f'layer{l}']['F3'])
    wes = jnp.concatenate([
        jnp.pad(params[f'layer{l}']['We'],
                ((0, 0), (0, HPAD - DIMS[l + 1])))
        for l in range(NL)], axis=1)

    def wmf(l, cols):
        w = params[f'layer{l}']['Wm']
        return jnp.pad(w, ((0, (HPAD if l > 0 else INPUT_DIM) - DIMS[l]),
                           (0, cols - DIMS[l + 1])))

    def wsf(l, cols):
        w = params[f'layer{l}']['Ws']
        return jnp.pad(w, ((0, (HPAD if l > 0 else INPUT_DIM) - DIMS[l]),
                           (0, cols - DIMS[l + 1])))

    vec16 = _sc_edge_vec(pos16, src, dst)

    edouts = _edge_dense(vec16, edge_attr, f1s, f2s, f3s, wes)
    eaf16, emb16 = edouts[0], edouts[1]
    planes = edouts[2:]

    h0, hw0, hw1, hw2 = _c0(x, w1f, w2f, wmf(0, HPAD))
    hws = [hw0, hw1, hw2]
    h = h0
    for l in range(NL - 1):
        aggs = _sc_message(l, src, dst,
                           planes[POFF[l]:POFF[l] + NSLICES[l]], hws)
        aggs = [a.reshape(2, N, SLICE) for a in aggs]
        cols = HPAD if l < NL - 2 else SLICE
        outs = _cl(l + 1, h, node_attr, aggs, wsf(l, HPAD), wmf(l + 1, cols))
        h = outs[0]
        hws = list(outs[1:])

    aggs = _sc_message(NL - 1, src, dst,
                       planes[POFF[NL - 1]:POFF[NL - 1] + 1], hws)
    out = _c4(h, node_attr, aggs[0].reshape(2, N, SLICE), wsf(NL - 1, DIMS[NL]))

    eaf = eaf16[:, 0:EDGE_ATTR_DIM]
    emb = emb16[:, 0:NB]
    batch = jnp.zeros((N,), dtype=jnp.int32)
    return (out, node_attr, src, dst, eaf, emb, batch)
